# baseline (speedup 1.0000x reference)
import jax
import jax.numpy as jnp
from jax import lax
from jax.experimental import pallas as pl
from jax.experimental.pallas import tpu as pltpu
from jax.experimental.pallas import tpu_sc as plsc

_L = 16
_NC = 2
_NS = 16
_NW = _NC * _NS


def _router_gather(router_hbm, x_hbm, out_hbm, table_v, idx_v, out_v):
    wid = lax.axis_index("s") * _NC + lax.axis_index("c")
    bpw = idx_v.shape[0]
    base = wid * bpw
    pltpu.sync_copy(router_hbm, table_v)
    pltpu.sync_copy(x_hbm.at[pl.ds(base, bpw)], idx_v)

    def body(i, carry):
        off = pl.multiple_of(i * _L, _L)
        ids = idx_v[pl.ds(off, _L)]
        out_v[pl.ds(off, _L)] = plsc.load_gather(table_v, [ids])
        return carry

    lax.fori_loop(0, bpw // _L, body, 0)
    pltpu.sync_copy(out_v, out_hbm.at[pl.ds(base, bpw)])


def kernel(x, router):
    B = x.size
    bpw = B // _NW
    vpad = ((router.shape[0] + _L - 1) // _L) * _L
    router_p = jnp.pad(router, (0, vpad - router.shape[0]))
    xf = x.reshape(-1)
    mesh = plsc.VectorSubcoreMesh(core_axis_name="c", subcore_axis_name="s")
    out = pl.kernel(
        _router_gather,
        out_type=jax.ShapeDtypeStruct((B,), jnp.int32),
        mesh=mesh,
        compiler_params=pltpu.CompilerParams(needs_layout_passes=False),
        scratch_types=[
            pltpu.VMEM((vpad,), jnp.int32),
            pltpu.VMEM((bpw,), jnp.int32),
            pltpu.VMEM((bpw,), jnp.int32),
        ],
    )(router_p, xf)
    return out.reshape(x.shape)


# speedup vs baseline: 1.1888x; 1.1888x over previous
import jax
import jax.numpy as jnp
from jax import lax
from jax.experimental import pallas as pl
from jax.experimental.pallas import tpu as pltpu
from jax.experimental.pallas import tpu_sc as plsc

_NC = 2
_NS = 16
_NW = _NC * _NS
_C = 128
_RPW = 8


def _router_gather(router_hbm, x_hbm, out_hbm, idx_v, out_v, sem):
    wid = lax.axis_index("s") * _NC + lax.axis_index("c")
    base = wid * _RPW
    pltpu.sync_copy(x_hbm.at[pl.ds(base, _RPW)], idx_v)
    copies = [
        pltpu.async_copy(router_hbm.at[idx_v.at[j]], out_v.at[j], sem)
        for j in range(_RPW)
    ]
    for c in copies:
        c.wait()
    pltpu.sync_copy(out_v, out_hbm.at[pl.ds(base, _RPW)])


def kernel(x, router):
    B = x.size
    xf = x.reshape(_NW * _RPW, _C)
    mesh = plsc.VectorSubcoreMesh(core_axis_name="c", subcore_axis_name="s")
    out = pl.kernel(
        _router_gather,
        out_type=jax.ShapeDtypeStruct((_NW * _RPW, _C), jnp.int32),
        mesh=mesh,
        compiler_params=pltpu.CompilerParams(needs_layout_passes=False),
        scratch_types=[
            pltpu.VMEM((_RPW, _C), jnp.int32),
            pltpu.VMEM((_RPW, _C), jnp.int32),
            pltpu.SemaphoreType.DMA,
        ],
    )(router, xf)
    return out.reshape(x.shape)


# speedup vs baseline: 1.2078x; 1.0160x over previous
import jax
import jax.numpy as jnp
from jax import lax
from jax.experimental import pallas as pl
from jax.experimental.pallas import tpu as pltpu
from jax.experimental.pallas import tpu_sc as plsc

_NC = 2
_NS = 16
_NW = _NC * _NS


def _router_gather(router_hbm, x_hbm, out_hbm, idx_v, out_v, sem):
    wid = lax.axis_index("s") * _NC + lax.axis_index("c")
    bpw = idx_v.shape[0]
    base = wid * bpw
    pltpu.sync_copy(x_hbm.at[pl.ds(base, bpw)], idx_v)
    pltpu.async_copy(router_hbm.at[idx_v], out_v, sem).wait()
    pltpu.sync_copy(out_v, out_hbm.at[pl.ds(base, bpw)])


def kernel(x, router):
    B = x.size
    bpw = B // _NW
    xf = x.reshape(B)
    mesh = plsc.VectorSubcoreMesh(core_axis_name="c", subcore_axis_name="s")
    out = pl.kernel(
        _router_gather,
        out_type=jax.ShapeDtypeStruct((B,), jnp.int32),
        mesh=mesh,
        compiler_params=pltpu.CompilerParams(needs_layout_passes=False),
        scratch_types=[
            pltpu.VMEM((bpw,), jnp.int32),
            pltpu.VMEM((bpw,), jnp.int32),
            pltpu.SemaphoreType.DMA,
        ],
    )(router, xf)
    return out.reshape(x.shape)
